# Initial kernel scaffold; baseline (speedup 1.0000x reference)
import jax
import jax.numpy as jnp
from jax.experimental import pallas as pl


def kernel(features, base_features, support_labels):
    raise NotImplementedError("write your pallas kernel here")


# speedup vs baseline: 3.9400x; 3.9400x over previous
import jax
import jax.numpy as jnp
from jax.experimental import pallas as pl

D = 512
BR = 2048


def _support_mean_kernel(sf_ref, q_ref):
    f = sf_ref[...]
    n2 = jnp.sum(f * f, axis=2, keepdims=True)
    fn = f * (1.0 / jnp.maximum(jnp.sqrt(n2), 1e-12))
    m = jnp.mean(fn, axis=1)
    mn = jnp.sqrt(jnp.sum(m * m, axis=1, keepdims=True))
    qn = m / jnp.maximum(mn, 1e-12)
    e = qn.shape[0]
    qe = jnp.concatenate(
        [qn, jnp.ones((1, D), jnp.float32), jnp.zeros((15 - e, D), jnp.float32)],
        axis=0,
    )
    q_ref[...] = qe


def _sim_kernel(n_base, bf_ref, q_ref, s_ref, g_ref):
    i = pl.program_id(0)
    x = bf_ref[...]
    e = s_ref.shape[0]
    pa = jax.lax.dot_general(
        q_ref[...], x, (((1,), (1,)), ((), ())), preferred_element_type=jnp.float32
    )
    p = pa[0:e, :]
    n2 = jax.lax.dot_general(
        jnp.ones((1, D), jnp.float32) , x * x, (((1,), (1,)), ((), ())),
        preferred_element_type=jnp.float32,
    )
    rn = 1.0 / jnp.maximum(jnp.sqrt(n2), 1e-12)
    s = p * rn
    col = jax.lax.broadcasted_iota(jnp.int32, (e, BR), 1) + i * BR
    s_ref[...] = jnp.where(col < n_base, s, -jnp.inf)
    g_ref[...] = rn


def _threshold_kernel(k_top, s_ref, t_ref):
    s = s_ref[...]
    b = jax.lax.bitcast_convert_type(s, jnp.int32)
    u = jnp.where(b < 0, ~b, b | jnp.int32(-(2**31))).astype(jnp.uint32)
    e = s.shape[0]

    def body(i, prefix):
        bit = jax.lax.shift_left(jnp.uint32(1), (31 - i).astype(jnp.uint32))
        cand = prefix | bit
        cnt = jnp.sum((u >= cand).astype(jnp.int32), axis=1, keepdims=True)
        return jnp.where(cnt >= k_top, cand, prefix)

    prefix = jax.lax.fori_loop(0, 32, body, jnp.zeros((e, 1), jnp.uint32))
    pi = prefix.astype(jnp.int32)
    fb = jnp.where(pi < 0, pi & jnp.int32(2**31 - 1), ~pi)
    t = jax.lax.bitcast_convert_type(fb, jnp.float32)
    t_ref[...] = jnp.broadcast_to(t, t_ref.shape)


def _combine_kernel(bf_ref, s_ref, g_ref, t_ref, a_ref):
    @pl.when(pl.program_id(0) == 0)
    def _():
        a_ref[...] = jnp.zeros_like(a_ref)

    x = bf_ref[...]
    s = s_ref[...]
    t = t_ref[:, 0:1]
    w = jnp.where(s >= t, jnp.sqrt(s), 0.0) * g_ref[...]
    a_ref[...] += jax.lax.dot_general(
        w, x, (((1,), (0,)), ((), ())), preferred_element_type=jnp.float32
    )


def _project_kernel(f_ref, a_ref, o_ref):
    f = f_ref[0]
    a = a_ref[0]
    an = jnp.sqrt(jnp.sum(a * a, axis=1, keepdims=True))
    ah = a / jnp.maximum(an, 1e-12)
    n2 = jnp.sum(f * f, axis=1, keepdims=True)
    fn = f * (1.0 / jnp.maximum(jnp.sqrt(n2), 1e-12))
    cos = jnp.sum(fn * ah, axis=1, keepdims=True)
    r = fn - cos * ah
    rn2 = jnp.sum(r * r, axis=1, keepdims=True)
    o_ref[0] = r / jnp.maximum(jnp.sqrt(rn2), 1e-12)


def kernel(features, base_features, support_labels):
    import functools

    e, rows, d = features.shape
    n_base = base_features.shape[0]
    n_support = support_labels.shape[1]
    k_top = 15000 if n_base >= 15000 else int(0.1 * n_base)

    n_pad = ((n_base + BR - 1) // BR) * BR
    nblk = n_pad // BR

    sf = features[:, :n_support, :]
    qext = pl.pallas_call(
        _support_mean_kernel,
        out_shape=jax.ShapeDtypeStruct((16, d), jnp.float32),
    )(sf)

    bfp = jnp.pad(base_features, ((0, n_pad - n_base), (0, 0)))

    s, g = pl.pallas_call(
        functools.partial(_sim_kernel, n_base),
        grid=(nblk,),
        in_specs=[
            pl.BlockSpec((BR, d), lambda i: (i, 0)),
            pl.BlockSpec((16, d), lambda i: (0, 0)),
        ],
        out_specs=[
            pl.BlockSpec((e, BR), lambda i: (0, i)),
            pl.BlockSpec((1, BR), lambda i: (0, i)),
        ],
        out_shape=[
            jax.ShapeDtypeStruct((e, n_pad), jnp.float32),
            jax.ShapeDtypeStruct((1, n_pad), jnp.float32),
        ],
    )(bfp, qext)

    t = pl.pallas_call(
        functools.partial(_threshold_kernel, k_top),
        out_shape=jax.ShapeDtypeStruct((e, 128), jnp.float32),
    )(s)

    a = pl.pallas_call(
        _combine_kernel,
        grid=(nblk,),
        in_specs=[
            pl.BlockSpec((BR, d), lambda i: (i, 0)),
            pl.BlockSpec((e, BR), lambda i: (0, i)),
            pl.BlockSpec((1, BR), lambda i: (0, i)),
            pl.BlockSpec((e, 128), lambda i: (0, 0)),
        ],
        out_specs=pl.BlockSpec((e, d), lambda i: (0, 0)),
        out_shape=jax.ShapeDtypeStruct((e, d), jnp.float32),
    )(bfp, s, g, t)

    out = pl.pallas_call(
        _project_kernel,
        grid=(e,),
        in_specs=[
            pl.BlockSpec((1, rows, d), lambda i: (i, 0, 0)),
            pl.BlockSpec((1, 1, d), lambda i: (i, 0, 0)),
        ],
        out_specs=pl.BlockSpec((1, rows, d), lambda i: (i, 0, 0)),
        out_shape=jax.ShapeDtypeStruct((e, rows, d), jnp.float32),
    )(features, a.reshape(e, 1, d))

    return out
